# Initial kernel scaffold; baseline (speedup 1.0000x reference)
import jax
import jax.numpy as jnp
from jax.experimental import pallas as pl


def kernel(x, target):
    raise NotImplementedError("write your pallas kernel here")


# speedup vs baseline: 1.6678x; 1.6678x over previous
import jax
import jax.numpy as jnp
import numpy as np
from jax.experimental import pallas as pl
from jax.experimental.pallas import tpu as pltpu

ROWS, COLS = 16384, 1000
BLOCK_R = 512
NBLK = ROWS // BLOCK_R

_THR = tuple(float(np.float32(np.log(i / (10.0 - i)))) for i in range(1, 10))


def _hist_kernel(x_ref, t_ref, o_ref):
    x = x_ref[...]
    tgt = t_ref[...]
    col = jax.lax.broadcasted_iota(jnp.int32, (BLOCK_R, COLS), 1)
    is_t = col == tgt
    xe = jnp.where(is_t, -x, x)
    lane = jax.lax.broadcasted_iota(jnp.int32, (1, 16), 1)
    sums = jnp.zeros((1, 16), jnp.float32)
    for i, thr in enumerate(_THR):
        s = jnp.sum((xe >= thr).astype(jnp.float32))
        sums = jnp.where(lane == i, s, sums)

    @pl.when(pl.program_id(0) == 0)
    def _init():
        o_ref[...] = sums

    @pl.when(pl.program_id(0) != 0)
    def _acc():
        o_ref[...] += sums


def _loss_kernel(x_ref, t_ref, logc_ref, o_ref):
    x = x_ref[...]
    tgt = t_ref[...]
    col = jax.lax.broadcasted_iota(jnp.int32, (BLOCK_R, COLS), 1)
    is_t = col == tgt
    xe = jnp.where(is_t, -x, x)
    logcg = jnp.full_like(x, logc_ref[0])
    for i, thr in enumerate(_THR):
        logcg = jnp.where(xe >= thr, logc_ref[i + 1], logcg)
    t_row = jnp.sum(jnp.where(is_t, x, 0.0), axis=1, keepdims=True)
    logct = jnp.sum(jnp.where(is_t, logcg, 0.0), axis=1, keepdims=True)
    wx = x + jnp.minimum(logcg - logct, 0.0)
    m = jnp.max(wx, axis=1, keepdims=True)
    lse = m + jnp.log(jnp.sum(jnp.exp(wx - m), axis=1, keepdims=True))
    part = jnp.sum(lse - t_row)

    @pl.when(pl.program_id(0) == 0)
    def _init():
        o_ref[0, 0] = part

    @pl.when(pl.program_id(0) != 0)
    def _acc():
        o_ref[0, 0] += part


def kernel(x, target):
    t2 = target.astype(jnp.int32).reshape(ROWS, 1)

    hist = pl.pallas_call(
        _hist_kernel,
        grid=(NBLK,),
        in_specs=[
            pl.BlockSpec((BLOCK_R, COLS), lambda i: (i, 0)),
            pl.BlockSpec((BLOCK_R, 1), lambda i: (i, 0)),
        ],
        out_specs=pl.BlockSpec((1, 16), lambda i: (0, 0)),
        out_shape=jax.ShapeDtypeStruct((1, 16), jnp.float32),
    )(x, t2)

    s = hist[0, :9]
    total = np.float32(ROWS * COLS)
    counts = jnp.concatenate([
        jnp.array([total]) - s[:1],
        s[:8] - s[1:9],
        s[8:9],
    ])
    logc = jnp.log(counts)
    logc16 = jnp.pad(logc, (0, 6)).astype(jnp.float32)

    loss_sum = pl.pallas_call(
        _loss_kernel,
        grid=(NBLK,),
        in_specs=[
            pl.BlockSpec((BLOCK_R, COLS), lambda i: (i, 0)),
            pl.BlockSpec((BLOCK_R, 1), lambda i: (i, 0)),
            pl.BlockSpec(memory_space=pltpu.SMEM),
        ],
        out_specs=pl.BlockSpec((1, 1), lambda i: (0, 0),
                               memory_space=pltpu.SMEM),
        out_shape=jax.ShapeDtypeStruct((1, 1), jnp.float32),
    )(x, t2, logc16)

    return loss_sum[0, 0] / total * np.float32(COLS)


# speedup vs baseline: 1.7425x; 1.0448x over previous
import jax
import jax.numpy as jnp
import numpy as np
from jax.experimental import pallas as pl
from jax.experimental.pallas import tpu as pltpu

ROWS, COLS = 16384, 1000
BLOCK_R = 512
NBLK = ROWS // BLOCK_R
TOTAL = float(ROWS * COLS)

_THR = tuple(float(np.float32(np.log(i / (10.0 - i)))) for i in range(1, 10))


def _fused_kernel(x_ref, t_ref, o_ref, d_ref, cnt_ref):
    pid = pl.program_id(0)
    x = x_ref[...]
    tgt = t_ref[...]
    col = jax.lax.broadcasted_iota(jnp.int32, (BLOCK_R, COLS), 1)
    is_t = col == tgt
    xe = jnp.where(is_t, -x, x)
    ex = jnp.exp(x)

    parts = [jnp.sum(ex, axis=1, keepdims=True)]
    lane = jax.lax.broadcasted_iota(jnp.int32, (1, 16), 1)
    cvec = jnp.zeros((1, 16), jnp.float32)
    for k, thr in enumerate(_THR):
        m = xe >= thr
        parts.append(jnp.sum(jnp.where(m, ex, 0.0), axis=1, keepdims=True))
        cvec = jnp.where(lane == k, jnp.sum(m.astype(jnp.float32)), cvec)
    parts.append(jnp.sum(jnp.where(is_t, x, 0.0), axis=1, keepdims=True))
    block = jnp.concatenate(parts, axis=1)
    d_ref[pl.ds(pid * BLOCK_R, BLOCK_R), 0:11] = block

    @pl.when(pid == 0)
    def _init():
        cnt_ref[...] = cvec

    @pl.when(pid != 0)
    def _acc():
        cnt_ref[...] += cvec

    @pl.when(pid == NBLK - 1)
    def _finalize():
        s = cnt_ref[...]
        c = jnp.concatenate([
            jnp.full((1, 1), TOTAL, jnp.float32) - s[:, 0:1],
            s[:, 0:8] - s[:, 1:9],
            s[:, 8:9],
        ], axis=1)
        D = d_ref[:, 0:10]
        trow = d_ref[:, 10:11]
        dnext = jnp.concatenate(
            [D[:, 1:10], jnp.zeros((ROWS, 1), jnp.float32)], axis=1)
        e_bins = D - dnext
        nt = -trow
        cbt = jnp.zeros((ROWS, 1), jnp.float32) + c[:, 0:1]
        for k, thr in enumerate(_THR):
            cbt = jnp.where(nt >= thr, c[:, k + 1:k + 2], cbt)
        w = jnp.minimum(c * (1.0 / cbt), 1.0)
        sw = jnp.sum(w * e_bins, axis=1, keepdims=True)
        o_ref[0, 0] = jnp.sum(jnp.log(sw) - trow) / np.float32(ROWS)


def kernel(x, target):
    t2 = target.astype(jnp.int32).reshape(ROWS, 1)

    loss = pl.pallas_call(
        _fused_kernel,
        grid=(NBLK,),
        in_specs=[
            pl.BlockSpec((BLOCK_R, COLS), lambda i: (i, 0)),
            pl.BlockSpec((BLOCK_R, 1), lambda i: (i, 0)),
        ],
        out_specs=pl.BlockSpec((1, 1), lambda i: (0, 0),
                               memory_space=pltpu.SMEM),
        out_shape=jax.ShapeDtypeStruct((1, 1), jnp.float32),
        scratch_shapes=[
            pltpu.VMEM((ROWS, 16), jnp.float32),
            pltpu.VMEM((1, 16), jnp.float32),
        ],
    )(x, t2)

    return loss[0, 0]
